# Initial kernel scaffold; baseline (speedup 1.0000x reference)
import jax
import jax.numpy as jnp
from jax.experimental import pallas as pl


def kernel(x, gate_W, gate_b, expert_W1, expert_b1, expert_W2, expert_b2):
    raise NotImplementedError("write your pallas kernel here")


# speedup vs baseline: 3.2927x; 3.2927x over previous
import functools

import jax
import jax.numpy as jnp
from jax import lax
from jax.experimental import pallas as pl
from jax.experimental.pallas import tpu as pltpu
from jax.experimental.pallas import tpu_sc as plsc

N_EXPERTS = 8
TOP_K = 2
BLK = 256
H_CHUNKS = 1
LANES = 128
SC_CORES = 2
SC_SUBCORES = 16
NW = SC_CORES * SC_SUBCORES


def _gate_body(x_ref, w_ref, b_ref, out_ref):
    x = x_ref[...]
    logits = jnp.dot(x, w_ref[...], preferred_element_type=jnp.float32)
    logits = logits + b_ref[...]
    n = x.shape[0]
    lane = lax.broadcasted_iota(jnp.int32, (n, LANES), 1)
    valid = lane < N_EXPERTS
    neg = jnp.float32(-1e30)
    lm = jnp.where(valid, logits, neg)
    m1 = jnp.max(lm, axis=1, keepdims=True)
    i1 = jnp.min(jnp.where(lm == m1, lane, LANES), axis=1, keepdims=True)
    l2 = jnp.where(lane == i1, neg, lm)
    m2 = jnp.max(l2, axis=1, keepdims=True)
    i2 = jnp.min(jnp.where(l2 == m2, lane, LANES), axis=1, keepdims=True)
    e = jnp.exp(m2 - m1)
    d = 1.0 + e
    w1 = 1.0 / d
    w2 = e / d
    out = jnp.where(lane == 0, w1,
          jnp.where(lane == 1, w2,
          jnp.where(lane == 2, i1.astype(jnp.float32),
          jnp.where(lane == 3, i2.astype(jnp.float32), 0.0))))
    out_ref[...] = out


def _ffn_body(be_ref, xs_ref, w1_ref, b1_ref, w2_ref, b2_ref, wt_ref,
              out_ref, acc_ref):
    hc = pl.program_id(1)
    x = xs_ref[...]
    h = jnp.dot(x, w1_ref[0], preferred_element_type=jnp.float32)
    h = h + b1_ref[0]
    h = 0.5 * h * (1.0 + lax.erf(h * jnp.float32(0.7071067811865476)))
    part = jnp.dot(h, w2_ref[0], preferred_element_type=jnp.float32)

    @pl.when(hc == 0)
    def _():
        acc_ref[...] = part

    @pl.when(hc > 0)
    def _():
        acc_ref[...] = acc_ref[...] + part

    @pl.when(hc == H_CHUNKS - 1)
    def _():
        out_ref[...] = (acc_ref[...] + b2_ref[0]) * wt_ref[...]


def _sc_gather(x_hbm, idx_hbm, out_hbm, idx_v, rows_v, sem, *, rows_per_chunk,
               chunks):
    wid = lax.axis_index("s") * SC_CORES + lax.axis_index("c")
    pltpu.sync_copy(idx_hbm.at[pl.ds(wid * chunks, chunks)], idx_v)
    for ch in range(chunks):
        pltpu.async_copy(x_hbm.at[idx_v.at[ch]], rows_v, sem).wait()
        base = (wid * chunks + ch) * rows_per_chunk
        pltpu.sync_copy(rows_v, out_hbm.at[pl.ds(base, rows_per_chunk)])


def _sc_combine(y_hbm, p0_hbm, p1_hbm, out_hbm, i0_v, i1_v, buf0, buf1,
                sem0, sem1, *, tokens_per_worker, c_dim):
    wid = lax.axis_index("s") * SC_CORES + lax.axis_index("c")
    pltpu.sync_copy(p0_hbm.at[pl.ds(wid, 1)], i0_v)
    pltpu.sync_copy(p1_hbm.at[pl.ds(wid, 1)], i1_v)
    d0 = pltpu.async_copy(y_hbm.at[i0_v.at[0]], buf0, sem0)
    d1 = pltpu.async_copy(y_hbm.at[i1_v.at[0]], buf1, sem1)
    d0.wait()
    d1.wait()
    nseg = c_dim // 16

    def body(r, carry):
        for cseg in range(nseg):
            s = buf0[r, pl.ds(cseg * 16, 16)] + buf1[r, pl.ds(cseg * 16, 16)]
            buf0[r, pl.ds(cseg * 16, 16)] = s
        return carry

    lax.fori_loop(0, tokens_per_worker, body, 0)
    pltpu.sync_copy(buf0, out_hbm.at[pl.ds(wid * tokens_per_worker,
                                           tokens_per_worker)])


def kernel(x, gate_W, gate_b, expert_W1, expert_b1, expert_W2, expert_b2):
    B, T, C = x.shape
    E = expert_W1.shape[0]
    H = expert_W1.shape[2]
    N = B * T
    NP = N * TOP_K
    NPAD = NP + E * BLK
    NBLK = NPAD // BLK
    x_flat = x.reshape(N, C)

    gw_p = jnp.zeros((C, LANES), jnp.float32).at[:, :E].set(gate_W)
    gb_p = jnp.zeros((1, LANES), jnp.float32).at[0, :E].set(gate_b)
    gate_out = pl.pallas_call(
        _gate_body,
        out_shape=jax.ShapeDtypeStruct((N, LANES), jnp.float32),
    )(x_flat, gw_p, gb_p)
    w_flat = gate_out[:, :TOP_K].reshape(-1)
    e_flat = gate_out[:, 2:2 + TOP_K].astype(jnp.int32).reshape(-1)

    onehot = (e_flat[:, None] == jnp.arange(E)[None, :]).astype(jnp.int32)
    csum = jnp.cumsum(onehot, axis=0)
    rank = jnp.sum(csum * onehot, axis=1) - 1
    counts = csum[-1]
    nblk_e = (counts + BLK - 1) // BLK
    cblk = jnp.cumsum(nblk_e)
    padded_start = (cblk - nblk_e) * BLK
    slot = jnp.sum(onehot * padded_start[None, :], axis=1) + rank
    token_of_pair = jnp.arange(NP, dtype=jnp.int32) // TOP_K
    row_token = jnp.zeros((NPAD,), jnp.int32).at[slot].set(token_of_pair)
    w_sorted = jnp.zeros((NPAD,), jnp.float32).at[slot].set(w_flat)
    block_expert = jnp.minimum(
        jnp.searchsorted(cblk, jnp.arange(NBLK), side="right"),
        E - 1).astype(jnp.int32)
    pos = slot.reshape(N, TOP_K).astype(jnp.int32)

    rows_per_worker = NPAD // NW
    rows_per_chunk = 96 if rows_per_worker % 96 == 0 else rows_per_worker
    while rows_per_chunk * C * 4 > 400_000:
        rows_per_chunk //= 2
    chunks = rows_per_worker // rows_per_chunk
    idx2d = row_token.reshape(NW * chunks, rows_per_chunk)
    mesh = plsc.VectorSubcoreMesh(core_axis_name="c", subcore_axis_name="s",
                                  num_cores=SC_CORES, num_subcores=SC_SUBCORES)
    x_sorted = pl.kernel(
        functools.partial(_sc_gather, rows_per_chunk=rows_per_chunk,
                          chunks=chunks),
        mesh=mesh,
        out_type=jax.ShapeDtypeStruct((NPAD, C), jnp.float32),
        scratch_types=[
            pltpu.VMEM((chunks, rows_per_chunk), jnp.int32),
            pltpu.VMEM((rows_per_chunk, C), jnp.float32),
            pltpu.SemaphoreType.DMA,
        ],
    )(x_flat, idx2d)

    hch = H // H_CHUNKS
    grid_spec = pltpu.PrefetchScalarGridSpec(
        num_scalar_prefetch=1,
        grid=(NBLK, H_CHUNKS),
        in_specs=[
            pl.BlockSpec((BLK, C), lambda b, hc, be: (b, 0)),
            pl.BlockSpec((1, C, hch), lambda b, hc, be: (be[b], 0, hc)),
            pl.BlockSpec((1, 1, hch), lambda b, hc, be: (be[b], 0, hc)),
            pl.BlockSpec((1, hch, C), lambda b, hc, be: (be[b], hc, 0)),
            pl.BlockSpec((1, 1, C), lambda b, hc, be: (be[b], 0, 0)),
            pl.BlockSpec((BLK, 1), lambda b, hc, be: (b, 0)),
        ],
        out_specs=pl.BlockSpec((BLK, C), lambda b, hc, be: (b, 0)),
        scratch_shapes=[pltpu.VMEM((BLK, C), jnp.float32)],
    )
    y_sorted = pl.pallas_call(
        _ffn_body,
        grid_spec=grid_spec,
        out_shape=jax.ShapeDtypeStruct((NPAD, C), jnp.float32),
    )(block_expert, x_sorted, expert_W1, expert_b1.reshape(E, 1, H),
      expert_W2, expert_b2.reshape(E, 1, C), w_sorted.reshape(NPAD, 1))

    tpw = N // NW
    p0 = pos[:, 0].reshape(NW, tpw)
    p1 = pos[:, 1].reshape(NW, tpw)
    out_flat = pl.kernel(
        functools.partial(_sc_combine, tokens_per_worker=tpw, c_dim=C),
        mesh=mesh,
        out_type=jax.ShapeDtypeStruct((N, C), jnp.float32),
        scratch_types=[
            pltpu.VMEM((1, tpw), jnp.int32),
            pltpu.VMEM((1, tpw), jnp.int32),
            pltpu.VMEM((tpw, C), jnp.float32),
            pltpu.VMEM((tpw, C), jnp.float32),
            pltpu.SemaphoreType.DMA,
            pltpu.SemaphoreType.DMA,
        ],
    )(y_sorted, p0, p1)

    return out_flat.reshape(B, T, C)


# speedup vs baseline: 5.2084x; 1.5818x over previous
import functools

import jax
import jax.numpy as jnp
from jax import lax
from jax.experimental import pallas as pl
from jax.experimental.pallas import tpu as pltpu
from jax.experimental.pallas import tpu_sc as plsc

N_EXPERTS = 8
TOP_K = 2
BLK = 128
BLK_LOG2 = 7
H_CHUNKS = 1
LANES = 128
SC_CORES = 2
SC_SUBCORES = 16
NW = SC_CORES * SC_SUBCORES
SCL = 16
WCOL = 128


def _gate_body(lg_ref, out_ref):
    lm = lg_ref[...]
    n = lm.shape[0]
    lane = lax.broadcasted_iota(jnp.int32, (n, N_EXPERTS), 1)
    neg = jnp.float32(-1e30)
    m1 = jnp.max(lm, axis=1, keepdims=True)
    i1 = jnp.min(jnp.where(lm == m1, lane, N_EXPERTS), axis=1, keepdims=True)
    l2 = jnp.where(lane == i1, neg, lm)
    m2 = jnp.max(l2, axis=1, keepdims=True)
    i2 = jnp.min(jnp.where(l2 == m2, lane, N_EXPERTS), axis=1, keepdims=True)
    e = jnp.exp(m2 - m1)
    d = 1.0 + e
    w1 = 1.0 / d
    w2 = e / d
    out = jnp.where(lane == 0, w1,
          jnp.where(lane == 1, w2,
          jnp.where(lane == 2, i1.astype(jnp.float32),
          jnp.where(lane == 3, i2.astype(jnp.float32), 0.0))))
    out_ref[...] = out


def _ffn_body(be_ref, xs_ref, w1_ref, b1_ref, w2_ref, b2_ref, wt_ref,
              out_ref, acc_ref):
    hc = pl.program_id(1)
    x = xs_ref[...]
    h = jnp.dot(x, w1_ref[0], preferred_element_type=jnp.float32)
    h = h + b1_ref[0]
    h = 0.5 * h * (1.0 + lax.erf(h * jnp.float32(0.7071067811865476)))
    part = jnp.dot(h, w2_ref[0], preferred_element_type=jnp.float32)

    @pl.when(hc == 0)
    def _():
        acc_ref[...] = part

    @pl.when(hc > 0)
    def _():
        acc_ref[...] = acc_ref[...] + part

    @pl.when(hc == H_CHUNKS - 1)
    def _():
        out_ref[...] = (acc_ref[...] + b2_ref[0]) * wt_ref[...][:, 0:1]


def _splat(vec, e):
    idx = jnp.full((SCL, 1), e, jnp.int32)
    dn = lax.GatherDimensionNumbers(offset_dims=(), collapsed_slice_dims=(0,),
                                    start_index_map=(0,))
    return lax.gather(vec, idx, dn, slice_sizes=(1,),
                      mode=lax.GatherScatterMode.PROMISE_IN_BOUNDS)


def _sc_route(x_hbm, eid0_hbm, eid1_hbm, wgt0_hbm, wgt1_hbm, xs_hbm,
              wcol_hbm, p0_hbm, p1_hbm, bexp_hbm, eid0_v, eid1_v, wgt0_v,
              wgt1_v, xbuf, slot2d, w16, bexp_v, sem_x, sem_e, sem_w, s0, s1,
              s2, *, tok_pw, nrow, nblk, nblk_pad):
    wid = lax.axis_index("s") * SC_CORES + lax.axis_index("c")
    izero = jnp.zeros((SCL,), jnp.int32)
    ione = jnp.ones((SCL,), jnp.int32)
    iota = lax.iota(jnp.int32, SCL)

    dx = pltpu.async_copy(x_hbm.at[pl.ds(wid * tok_pw, tok_pw)], xbuf, sem_x)
    pltpu.async_copy(eid0_hbm, eid0_v, sem_e).wait()
    pltpu.async_copy(eid1_hbm, eid1_v, sem_e).wait()
    dw0 = pltpu.async_copy(wgt0_hbm, wgt0_v, sem_w)
    dw1 = pltpu.async_copy(wgt1_hbm, wgt1_v, sem_w)

    def count_body(j, acc):
        ev0 = eid0_v[j]
        ev1 = eid1_v[j]
        for e in range(N_EXPERTS):
            c = (plsc.all_reduce_population_count(ev0 == e)
                 + plsc.all_reduce_population_count(ev1 == e))
            onehot = jnp.where(iota == e, ione, izero)
            acc = acc + c * onehot
        return acc

    rows_pt = tok_pw // SCL
    tot = lax.fori_loop(0, nrow, count_body, izero)
    pref = lax.fori_loop(0, wid * rows_pt, count_body, izero)

    nblk_e = lax.shift_right_logical(tot + (BLK - 1), BLK_LOG2)
    cblk = plsc.cumsum(nblk_e)
    pstart = (cblk - nblk_e) * BLK
    base = pstart + pref

    for c in range(nblk_pad // SCL):
        bvec = iota + c * SCL
        cnt = izero
        for e in range(N_EXPERTS):
            ce = _splat(cblk, e)
            cnt = cnt + jnp.where(bvec >= ce, ione, izero)
        bexp_v[pl.ds(c * SCL, SCL)] = jnp.minimum(cnt, N_EXPERTS - 1)

    run = [izero] * N_EXPERTS
    base_splat = [_splat(base, e) for e in range(N_EXPERTS)]
    for j in range(2 * rows_pt):
        k, jj = divmod(j, rows_pt)
        ev = (eid0_v if k == 0 else eid1_v)[wid * rows_pt + jj]
        slot = izero
        for e in range(N_EXPERTS):
            m = ev == e
            cs = plsc.cumsum(jnp.where(m, ione, izero))
            c = plsc.all_reduce_population_count(m)
            slot = jnp.where(m, base_splat[e] + run[e] + cs - 1, slot)
            run[e] = run[e] + c
        slot2d[k, pl.ds(jj * SCL, SCL)] = slot

    pltpu.sync_copy(slot2d.at[pl.ds(0, 1)], p0_hbm.at[pl.ds(wid, 1)])
    pltpu.sync_copy(slot2d.at[pl.ds(1, 1)], p1_hbm.at[pl.ds(wid, 1)])

    @pl.when(wid == 0)
    def _():
        pltpu.sync_copy(bexp_v, bexp_hbm)

    dx.wait()
    dw0.wait()
    dw1.wait()
    d0 = pltpu.async_copy(xbuf, xs_hbm.at[slot2d.at[0]], s0)
    d1 = pltpu.async_copy(xbuf, xs_hbm.at[slot2d.at[1]], s1)
    for k in range(TOP_K):
        wv_tab = wgt0_v if k == 0 else wgt1_v
        for jj in range(rows_pt):
            wv = wv_tab[wid * rows_pt + jj]
            plsc.store_scatter(w16, [jj * SCL + iota, izero], wv)
        pltpu.async_copy(w16, wcol_hbm.at[slot2d.at[k]], s2).wait()
    d0.wait()
    d1.wait()


def _sc_combine(y_hbm, p0_hbm, p1_hbm, out_hbm, i0_v, i1_v, buf0, buf1,
                sem0, sem1, *, tokens_per_worker, c_dim):
    wid = lax.axis_index("s") * SC_CORES + lax.axis_index("c")
    pltpu.sync_copy(p0_hbm.at[pl.ds(wid, 1)], i0_v)
    pltpu.sync_copy(p1_hbm.at[pl.ds(wid, 1)], i1_v)
    d0 = pltpu.async_copy(y_hbm.at[i0_v.at[0]], buf0, sem0)
    d1 = pltpu.async_copy(y_hbm.at[i1_v.at[0]], buf1, sem1)
    d0.wait()
    d1.wait()
    nseg = c_dim // SCL

    def body(r, carry):
        for cseg in range(nseg):
            s = buf0[r, pl.ds(cseg * SCL, SCL)] + buf1[r, pl.ds(cseg * SCL, SCL)]
            buf0[r, pl.ds(cseg * SCL, SCL)] = s
        return carry

    lax.fori_loop(0, tokens_per_worker, body, 0)
    pltpu.sync_copy(buf0, out_hbm.at[pl.ds(wid * tokens_per_worker,
                                           tokens_per_worker)])


def kernel(x, gate_W, gate_b, expert_W1, expert_b1, expert_W2, expert_b2):
    B, T, C = x.shape
    E = expert_W1.shape[0]
    H = expert_W1.shape[2]
    N = B * T
    NP = N * TOP_K
    NPAD = NP + E * BLK
    NBLK = NPAD // BLK
    NBLK_PAD = ((NBLK + SCL - 1) // SCL) * SCL
    x_flat = x.reshape(N, C)

    gate_logits = x_flat @ gate_W + gate_b
    gate_out = pl.pallas_call(
        _gate_body,
        out_shape=jax.ShapeDtypeStruct((N, N_EXPERTS), jnp.float32),
    )(gate_logits)
    tok_pw = N // NW
    nrow = N // SCL
    eid0 = gate_out[:, 2].astype(jnp.int32).reshape(nrow, SCL)
    eid1 = gate_out[:, 3].astype(jnp.int32).reshape(nrow, SCL)
    wgt0 = gate_out[:, 0].reshape(nrow, SCL)
    wgt1 = gate_out[:, 1].reshape(nrow, SCL)

    mesh = plsc.VectorSubcoreMesh(core_axis_name="c", subcore_axis_name="s",
                                  num_cores=SC_CORES, num_subcores=SC_SUBCORES)
    route = pl.kernel(
        functools.partial(_sc_route, tok_pw=tok_pw, nrow=nrow,
                          nblk=NBLK, nblk_pad=NBLK_PAD),
        mesh=mesh,
        compiler_params=pltpu.CompilerParams(needs_layout_passes=False),
        out_type=(
            jax.ShapeDtypeStruct((NPAD, C), jnp.float32),
            jax.ShapeDtypeStruct((NPAD, WCOL), jnp.float32),
            jax.ShapeDtypeStruct((NW, tok_pw), jnp.int32),
            jax.ShapeDtypeStruct((NW, tok_pw), jnp.int32),
            jax.ShapeDtypeStruct((NBLK_PAD,), jnp.int32),
        ),
        scratch_types=[
            pltpu.VMEM((nrow, SCL), jnp.int32),
            pltpu.VMEM((nrow, SCL), jnp.int32),
            pltpu.VMEM((nrow, SCL), jnp.float32),
            pltpu.VMEM((nrow, SCL), jnp.float32),
            pltpu.VMEM((tok_pw, C), jnp.float32),
            pltpu.VMEM((TOP_K, tok_pw), jnp.int32),
            pltpu.VMEM((tok_pw, WCOL), jnp.float32),
            pltpu.VMEM((NBLK_PAD,), jnp.int32),
            pltpu.SemaphoreType.DMA,
            pltpu.SemaphoreType.DMA,
            pltpu.SemaphoreType.DMA,
            pltpu.SemaphoreType.DMA,
            pltpu.SemaphoreType.DMA,
            pltpu.SemaphoreType.DMA,
        ],
    )(x_flat, eid0, eid1, wgt0, wgt1)
    x_sorted, wcol, p0, p1, block_expert = route

    hch = H // H_CHUNKS
    grid_spec = pltpu.PrefetchScalarGridSpec(
        num_scalar_prefetch=1,
        grid=(NBLK, H_CHUNKS),
        in_specs=[
            pl.BlockSpec((BLK, C), lambda b, hc, be: (b, 0)),
            pl.BlockSpec((1, C, hch), lambda b, hc, be: (be[b], 0, hc)),
            pl.BlockSpec((1, 1, hch), lambda b, hc, be: (be[b], 0, hc)),
            pl.BlockSpec((1, hch, C), lambda b, hc, be: (be[b], hc, 0)),
            pl.BlockSpec((1, 1, C), lambda b, hc, be: (be[b], 0, 0)),
            pl.BlockSpec((BLK, WCOL), lambda b, hc, be: (b, 0)),
        ],
        out_specs=pl.BlockSpec((BLK, C), lambda b, hc, be: (b, 0)),
        scratch_shapes=[pltpu.VMEM((BLK, C), jnp.float32)],
    )
    y_sorted = pl.pallas_call(
        _ffn_body,
        grid_spec=grid_spec,
        out_shape=jax.ShapeDtypeStruct((NPAD, C), jnp.float32),
    )(block_expert, x_sorted, expert_W1, expert_b1.reshape(E, 1, H),
      expert_W2, expert_b2.reshape(E, 1, C), wcol)

    out_flat = pl.kernel(
        functools.partial(_sc_combine, tokens_per_worker=tok_pw, c_dim=C),
        mesh=mesh,
        out_type=jax.ShapeDtypeStruct((N, C), jnp.float32),
        scratch_types=[
            pltpu.VMEM((1, tok_pw), jnp.int32),
            pltpu.VMEM((1, tok_pw), jnp.int32),
            pltpu.VMEM((tok_pw, C), jnp.float32),
            pltpu.VMEM((tok_pw, C), jnp.float32),
            pltpu.SemaphoreType.DMA,
            pltpu.SemaphoreType.DMA,
        ],
    )(y_sorted, p0, p1)

    return out_flat.reshape(B, T, C)


# speedup vs baseline: 5.2118x; 1.0006x over previous
import functools

import jax
import jax.numpy as jnp
from jax import lax
from jax.experimental import pallas as pl
from jax.experimental.pallas import tpu as pltpu
from jax.experimental.pallas import tpu_sc as plsc

N_EXPERTS = 8
TOP_K = 2
BLK = 128
BLK_LOG2 = 7
H_CHUNKS = 1
LANES = 128
SC_CORES = 2
SC_SUBCORES = 16
NW = SC_CORES * SC_SUBCORES
SCL = 16
WCOL = 128


def _gate_body(lg_ref, out_ref):
    lm = lg_ref[...]
    n = lm.shape[0]
    lane = lax.broadcasted_iota(jnp.int32, (n, N_EXPERTS), 1)
    neg = jnp.float32(-1e30)
    m1 = jnp.max(lm, axis=1, keepdims=True)
    i1 = jnp.min(jnp.where(lm == m1, lane, N_EXPERTS), axis=1, keepdims=True)
    l2 = jnp.where(lane == i1, neg, lm)
    m2 = jnp.max(l2, axis=1, keepdims=True)
    i2 = jnp.min(jnp.where(l2 == m2, lane, N_EXPERTS), axis=1, keepdims=True)
    e = jnp.exp(m2 - m1)
    d = 1.0 + e
    w1 = 1.0 / d
    w2 = e / d
    out = jnp.where(lane == 0, w1,
          jnp.where(lane == 1, w2,
          jnp.where(lane == 2, i1.astype(jnp.float32),
          jnp.where(lane == 3, i2.astype(jnp.float32), 0.0))))
    out_ref[...] = out


def _ffn_body(be_ref, xs_ref, w1_ref, b1_ref, w2_ref, b2_ref, wt_ref,
              out_ref, acc_ref):
    hc = pl.program_id(1)
    x = xs_ref[...]
    h = jnp.dot(x, w1_ref[0], preferred_element_type=jnp.float32)
    h = h + b1_ref[0]
    h = 0.5 * h * (1.0 + lax.erf(h * jnp.float32(0.7071067811865476)))
    part = jnp.dot(h, w2_ref[0], preferred_element_type=jnp.float32)

    @pl.when(hc == 0)
    def _():
        acc_ref[...] = part

    @pl.when(hc > 0)
    def _():
        acc_ref[...] = acc_ref[...] + part

    @pl.when(hc == H_CHUNKS - 1)
    def _():
        out_ref[...] = (acc_ref[...] + b2_ref[0]) * wt_ref[...][:, 0:1]


def _splat(vec, e):
    idx = jnp.full((SCL, 1), e, jnp.int32)
    dn = lax.GatherDimensionNumbers(offset_dims=(), collapsed_slice_dims=(0,),
                                    start_index_map=(0,))
    return lax.gather(vec, idx, dn, slice_sizes=(1,),
                      mode=lax.GatherScatterMode.PROMISE_IN_BOUNDS)


def _sc_route(x_hbm, eid0_hbm, eid1_hbm, wgt0_hbm, wgt1_hbm, xs_hbm,
              wcol_hbm, p0_hbm, p1_hbm, bexp_hbm, eid0_v, eid1_v, wgt0_v,
              wgt1_v, xbuf, slot2d, w16, bexp_v, sem_x, sem_e, sem_w, s0, s1,
              s2, *, tok_pw, nrow, nblk, nblk_pad):
    wid = lax.axis_index("s") * SC_CORES + lax.axis_index("c")
    izero = jnp.zeros((SCL,), jnp.int32)
    ione = jnp.ones((SCL,), jnp.int32)
    iota = lax.iota(jnp.int32, SCL)

    dx = pltpu.async_copy(x_hbm.at[pl.ds(wid * tok_pw, tok_pw)], xbuf, sem_x)
    pltpu.async_copy(eid0_hbm, eid0_v, sem_e).wait()
    pltpu.async_copy(eid1_hbm, eid1_v, sem_e).wait()
    dw0 = pltpu.async_copy(wgt0_hbm, wgt0_v, sem_w)
    dw1 = pltpu.async_copy(wgt1_hbm, wgt1_v, sem_w)

    def count_body(j, acc):
        ev0 = eid0_v[j]
        ev1 = eid1_v[j]
        for e in range(N_EXPERTS):
            c = (plsc.all_reduce_population_count(ev0 == e)
                 + plsc.all_reduce_population_count(ev1 == e))
            onehot = jnp.where(iota == e, ione, izero)
            acc = acc + c * onehot
        return acc

    rows_pt = tok_pw // SCL
    tot = lax.fori_loop(0, nrow, count_body, izero)
    pref = lax.fori_loop(0, wid * rows_pt, count_body, izero)

    nblk_e = lax.shift_right_logical(tot + (BLK - 1), BLK_LOG2)
    cblk = plsc.cumsum(nblk_e)
    pstart = (cblk - nblk_e) * BLK
    base = pstart + pref

    for c in range(nblk_pad // SCL):
        bvec = iota + c * SCL
        cnt = izero
        for e in range(N_EXPERTS):
            ce = _splat(cblk, e)
            cnt = cnt + jnp.where(bvec >= ce, ione, izero)
        bexp_v[pl.ds(c * SCL, SCL)] = jnp.minimum(cnt, N_EXPERTS - 1)

    run = [izero] * N_EXPERTS
    base_splat = [_splat(base, e) for e in range(N_EXPERTS)]
    for j in range(2 * rows_pt):
        k, jj = divmod(j, rows_pt)
        ev = (eid0_v if k == 0 else eid1_v)[wid * rows_pt + jj]
        slot = izero
        for e in range(N_EXPERTS):
            m = ev == e
            cs = plsc.cumsum(jnp.where(m, ione, izero))
            c = plsc.all_reduce_population_count(m)
            slot = jnp.where(m, base_splat[e] + run[e] + cs - 1, slot)
            run[e] = run[e] + c
        slot2d[k, pl.ds(jj * SCL, SCL)] = slot

    pltpu.sync_copy(slot2d.at[pl.ds(0, 1)], p0_hbm.at[pl.ds(wid, 1)])
    pltpu.sync_copy(slot2d.at[pl.ds(1, 1)], p1_hbm.at[pl.ds(wid, 1)])

    @pl.when(wid == 0)
    def _():
        pltpu.sync_copy(bexp_v, bexp_hbm)

    dx.wait()
    dw0.wait()
    dw1.wait()
    d0 = pltpu.async_copy(xbuf, xs_hbm.at[slot2d.at[0]], s0)
    d1 = pltpu.async_copy(xbuf, xs_hbm.at[slot2d.at[1]], s1)
    for k in range(TOP_K):
        wv_tab = wgt0_v if k == 0 else wgt1_v
        for jj in range(rows_pt):
            wv = wv_tab[wid * rows_pt + jj]
            plsc.store_scatter(w16, [jj * SCL + iota, izero], wv)
        pltpu.async_copy(w16, wcol_hbm.at[slot2d.at[k]], s2).wait()
    d0.wait()
    d1.wait()


def _sc_combine(y_hbm, p0_hbm, p1_hbm, out_hbm, i0_v, i1_v, buf0a, buf1a,
                buf0b, buf1b, sem0, sem1, sem2, sem3, soa, sob,
                *, tokens_per_worker, c_dim):
    wid = lax.axis_index("s") * SC_CORES + lax.axis_index("c")
    half = tokens_per_worker // 2
    pltpu.sync_copy(p0_hbm.at[pl.ds(wid, 1)], i0_v)
    pltpu.sync_copy(p1_hbm.at[pl.ds(wid, 1)], i1_v)
    da0 = pltpu.async_copy(y_hbm.at[i0_v.at[0, pl.ds(0, half)]], buf0a, sem0)
    da1 = pltpu.async_copy(y_hbm.at[i1_v.at[0, pl.ds(0, half)]], buf1a, sem1)
    db0 = pltpu.async_copy(y_hbm.at[i0_v.at[0, pl.ds(half, half)]], buf0b,
                           sem2)
    db1 = pltpu.async_copy(y_hbm.at[i1_v.at[0, pl.ds(half, half)]], buf1b,
                           sem3)
    nseg = c_dim // SCL

    def add_rows(b0, b1):
        def body(r, carry):
            for cseg in range(nseg):
                s = b0[r, pl.ds(cseg * SCL, SCL)] + b1[r, pl.ds(cseg * SCL,
                                                                SCL)]
                b0[r, pl.ds(cseg * SCL, SCL)] = s
            return carry
        lax.fori_loop(0, half, body, 0)

    da0.wait()
    da1.wait()
    add_rows(buf0a, buf1a)
    doa = pltpu.async_copy(buf0a,
                           out_hbm.at[pl.ds(wid * tokens_per_worker, half)],
                           soa)
    db0.wait()
    db1.wait()
    add_rows(buf0b, buf1b)
    dob = pltpu.async_copy(
        buf0b, out_hbm.at[pl.ds(wid * tokens_per_worker + half, half)], sob)
    doa.wait()
    dob.wait()


def kernel(x, gate_W, gate_b, expert_W1, expert_b1, expert_W2, expert_b2):
    B, T, C = x.shape
    E = expert_W1.shape[0]
    H = expert_W1.shape[2]
    N = B * T
    NP = N * TOP_K
    NPAD = NP + E * BLK
    NBLK = NPAD // BLK
    NBLK_PAD = ((NBLK + SCL - 1) // SCL) * SCL
    x_flat = x.reshape(N, C)

    gate_logits = x_flat @ gate_W + gate_b
    gate_out = pl.pallas_call(
        _gate_body,
        out_shape=jax.ShapeDtypeStruct((N, N_EXPERTS), jnp.float32),
    )(gate_logits)
    tok_pw = N // NW
    nrow = N // SCL
    eid0 = gate_out[:, 2].astype(jnp.int32).reshape(nrow, SCL)
    eid1 = gate_out[:, 3].astype(jnp.int32).reshape(nrow, SCL)
    wgt0 = gate_out[:, 0].reshape(nrow, SCL)
    wgt1 = gate_out[:, 1].reshape(nrow, SCL)

    mesh = plsc.VectorSubcoreMesh(core_axis_name="c", subcore_axis_name="s",
                                  num_cores=SC_CORES, num_subcores=SC_SUBCORES)
    route = pl.kernel(
        functools.partial(_sc_route, tok_pw=tok_pw, nrow=nrow,
                          nblk=NBLK, nblk_pad=NBLK_PAD),
        mesh=mesh,
        compiler_params=pltpu.CompilerParams(needs_layout_passes=False),
        out_type=(
            jax.ShapeDtypeStruct((NPAD, C), jnp.float32),
            jax.ShapeDtypeStruct((NPAD, WCOL), jnp.float32),
            jax.ShapeDtypeStruct((NW, tok_pw), jnp.int32),
            jax.ShapeDtypeStruct((NW, tok_pw), jnp.int32),
            jax.ShapeDtypeStruct((NBLK_PAD,), jnp.int32),
        ),
        scratch_types=[
            pltpu.VMEM((nrow, SCL), jnp.int32),
            pltpu.VMEM((nrow, SCL), jnp.int32),
            pltpu.VMEM((nrow, SCL), jnp.float32),
            pltpu.VMEM((nrow, SCL), jnp.float32),
            pltpu.VMEM((tok_pw, C), jnp.float32),
            pltpu.VMEM((TOP_K, tok_pw), jnp.int32),
            pltpu.VMEM((tok_pw, WCOL), jnp.float32),
            pltpu.VMEM((NBLK_PAD,), jnp.int32),
            pltpu.SemaphoreType.DMA,
            pltpu.SemaphoreType.DMA,
            pltpu.SemaphoreType.DMA,
            pltpu.SemaphoreType.DMA,
            pltpu.SemaphoreType.DMA,
            pltpu.SemaphoreType.DMA,
        ],
    )(x_flat, eid0, eid1, wgt0, wgt1)
    x_sorted, wcol, p0, p1, block_expert = route

    hch = H // H_CHUNKS
    grid_spec = pltpu.PrefetchScalarGridSpec(
        num_scalar_prefetch=1,
        grid=(NBLK, H_CHUNKS),
        in_specs=[
            pl.BlockSpec((BLK, C), lambda b, hc, be: (b, 0)),
            pl.BlockSpec((1, C, hch), lambda b, hc, be: (be[b], 0, hc)),
            pl.BlockSpec((1, 1, hch), lambda b, hc, be: (be[b], 0, hc)),
            pl.BlockSpec((1, hch, C), lambda b, hc, be: (be[b], hc, 0)),
            pl.BlockSpec((1, 1, C), lambda b, hc, be: (be[b], 0, 0)),
            pl.BlockSpec((BLK, WCOL), lambda b, hc, be: (b, 0)),
        ],
        out_specs=pl.BlockSpec((BLK, C), lambda b, hc, be: (b, 0)),
        scratch_shapes=[pltpu.VMEM((BLK, C), jnp.float32)],
    )
    y_sorted = pl.pallas_call(
        _ffn_body,
        grid_spec=grid_spec,
        out_shape=jax.ShapeDtypeStruct((NPAD, C), jnp.float32),
    )(block_expert, x_sorted, expert_W1, expert_b1.reshape(E, 1, H),
      expert_W2, expert_b2.reshape(E, 1, C), wcol)

    out_flat = pl.kernel(
        functools.partial(_sc_combine, tokens_per_worker=tok_pw, c_dim=C),
        mesh=mesh,
        out_type=jax.ShapeDtypeStruct((N, C), jnp.float32),
        scratch_types=[
            pltpu.VMEM((1, tok_pw), jnp.int32),
            pltpu.VMEM((1, tok_pw), jnp.int32),
            pltpu.VMEM((tok_pw // 2, C), jnp.float32),
            pltpu.VMEM((tok_pw // 2, C), jnp.float32),
            pltpu.VMEM((tok_pw // 2, C), jnp.float32),
            pltpu.VMEM((tok_pw // 2, C), jnp.float32),
            pltpu.SemaphoreType.DMA,
            pltpu.SemaphoreType.DMA,
            pltpu.SemaphoreType.DMA,
            pltpu.SemaphoreType.DMA,
            pltpu.SemaphoreType.DMA,
            pltpu.SemaphoreType.DMA,
        ],
    )(y_sorted, p0, p1)

    return out_flat.reshape(B, T, C)


# speedup vs baseline: 5.3113x; 1.0191x over previous
import functools

import jax
import jax.numpy as jnp
from jax import lax
from jax.experimental import pallas as pl
from jax.experimental.pallas import tpu as pltpu
from jax.experimental.pallas import tpu_sc as plsc

N_EXPERTS = 8
TOP_K = 2
BLK = 128
BLK_LOG2 = 7
H_CHUNKS = 1
LANES = 128
SC_CORES = 2
SC_SUBCORES = 16
NW = SC_CORES * SC_SUBCORES
SCL = 16
WCOL = 128


def _gate_body(lg_ref, out_ref):
    lm = lg_ref[...]
    n = lm.shape[0]
    lane = lax.broadcasted_iota(jnp.int32, (n, LANES), 1)
    neg = jnp.float32(-1e30)
    m1 = jnp.max(lm, axis=1, keepdims=True)
    i1 = jnp.min(jnp.where(lm == m1, lane, LANES), axis=1, keepdims=True)
    l2 = jnp.where(lane == i1, neg, lm)
    m2 = jnp.max(l2, axis=1, keepdims=True)
    i2 = jnp.min(jnp.where(l2 == m2, lane, LANES), axis=1, keepdims=True)
    e = jnp.exp(m2 - m1)
    d = 1.0 + e
    w1 = 1.0 / d
    w2 = e / d
    out = jnp.where(lane == 0, w1,
          jnp.where(lane == 1, w2,
          jnp.where(lane == 2, i1.astype(jnp.float32),
          jnp.where(lane == 3, i2.astype(jnp.float32), 0.0))))
    out_ref[...] = out


def _ffn_body(be_ref, xs_ref, w1_ref, b1_ref, w2_ref, b2_ref, wt_ref,
              out_ref, acc_ref):
    hc = pl.program_id(1)
    x = xs_ref[...]
    h = jnp.dot(x, w1_ref[0], preferred_element_type=jnp.float32)
    h = h + b1_ref[0]
    h = 0.5 * h * (1.0 + lax.erf(h * jnp.float32(0.7071067811865476)))
    part = jnp.dot(h, w2_ref[0], preferred_element_type=jnp.float32)

    @pl.when(hc == 0)
    def _():
        acc_ref[...] = part

    @pl.when(hc > 0)
    def _():
        acc_ref[...] = acc_ref[...] + part

    @pl.when(hc == H_CHUNKS - 1)
    def _():
        out_ref[...] = (acc_ref[...] + b2_ref[0]) * wt_ref[...][:, 0:1]


def _splat(vec, e):
    idx = jnp.full((SCL, 1), e, jnp.int32)
    dn = lax.GatherDimensionNumbers(offset_dims=(), collapsed_slice_dims=(0,),
                                    start_index_map=(0,))
    return lax.gather(vec, idx, dn, slice_sizes=(1,),
                      mode=lax.GatherScatterMode.PROMISE_IN_BOUNDS)


def _sc_route(x_hbm, eid_hbm, wgt_hbm, xs_hbm, wcol_hbm, p0_hbm, p1_hbm,
              bexp_hbm, eid_v, wgt_v, xbuf, slot2d, w16, bexp_v,
              sem_x, sem_e, sem_w, s0, s1, s2, *, tok_pw, n_chunks, nblk,
              nblk_pad):
    wid = lax.axis_index("s") * SC_CORES + lax.axis_index("c")
    izero = jnp.zeros((SCL,), jnp.int32)
    ione = jnp.ones((SCL,), jnp.int32)
    iota = lax.iota(jnp.int32, SCL)

    dx = pltpu.async_copy(x_hbm.at[pl.ds(wid * tok_pw, tok_pw)], xbuf, sem_x)
    pltpu.async_copy(eid_hbm, eid_v, sem_e).wait()
    dw = pltpu.async_copy(wgt_hbm, wgt_v, sem_w)

    def count_body(j, acc):
        ev = eid_v[j]
        for e in range(N_EXPERTS):
            c = plsc.all_reduce_population_count(ev == e)
            onehot = jnp.where(iota == e, ione, izero)
            acc = acc + c * onehot
        return acc

    my_first_chunk = wid * (2 * tok_pw // SCL)
    tot = lax.fori_loop(0, n_chunks, count_body, izero)
    pref = lax.fori_loop(0, my_first_chunk, count_body, izero)

    nblk_e = lax.shift_right_logical(tot + (BLK - 1), BLK_LOG2)
    cblk = plsc.cumsum(nblk_e)
    pstart = (cblk - nblk_e) * BLK
    base = pstart + pref

    for c in range(nblk_pad // SCL):
        bvec = iota + c * SCL
        cnt = izero
        for e in range(N_EXPERTS):
            ce = _splat(cblk, e)
            cnt = cnt + jnp.where(bvec >= ce, ione, izero)
        bexp_v[pl.ds(c * SCL, SCL)] = jnp.minimum(cnt, N_EXPERTS - 1)

    run = [izero] * N_EXPERTS
    base_splat = [_splat(base, e) for e in range(N_EXPERTS)]
    for j in range(2 * tok_pw // SCL):
        r = my_first_chunk + j
        ev = eid_v[r]
        slot = izero
        for e in range(N_EXPERTS):
            m = ev == e
            cs = plsc.cumsum(jnp.where(m, ione, izero))
            c = plsc.all_reduce_population_count(m)
            slot = jnp.where(m, base_splat[e] + run[e] + cs - 1, slot)
            run[e] = run[e] + c
        k, jj = divmod(j, tok_pw // SCL)
        slot2d[k, pl.ds(jj * SCL, SCL)] = slot

    pltpu.sync_copy(slot2d.at[pl.ds(0, 1)], p0_hbm.at[pl.ds(wid, 1)])
    pltpu.sync_copy(slot2d.at[pl.ds(1, 1)], p1_hbm.at[pl.ds(wid, 1)])

    @pl.when(wid == 0)
    def _():
        pltpu.sync_copy(bexp_v, bexp_hbm)

    dx.wait()
    dw.wait()
    d0 = pltpu.async_copy(xbuf, xs_hbm.at[slot2d.at[0]], s0)
    d1 = pltpu.async_copy(xbuf, xs_hbm.at[slot2d.at[1]], s1)
    for k in range(TOP_K):
        for jj in range(tok_pw // SCL):
            wv = wgt_v[wid * (2 * tok_pw // SCL) + k * (tok_pw // SCL) + jj]
            plsc.store_scatter(w16, [jj * SCL + iota, izero], wv)
        pltpu.async_copy(w16, wcol_hbm.at[slot2d.at[k]], s2).wait()
    d0.wait()
    d1.wait()


def _sc_combine(y_hbm, p0_hbm, p1_hbm, out_hbm, i0_v, i1_v, buf0a, buf1a,
                buf0b, buf1b, sem0, sem1, sem2, sem3, soa, sob,
                *, tokens_per_worker, c_dim):
    wid = lax.axis_index("s") * SC_CORES + lax.axis_index("c")
    half = tokens_per_worker // 2
    pltpu.sync_copy(p0_hbm.at[pl.ds(wid, 1)], i0_v)
    pltpu.sync_copy(p1_hbm.at[pl.ds(wid, 1)], i1_v)
    da0 = pltpu.async_copy(y_hbm.at[i0_v.at[0, pl.ds(0, half)]], buf0a, sem0)
    da1 = pltpu.async_copy(y_hbm.at[i1_v.at[0, pl.ds(0, half)]], buf1a, sem1)
    db0 = pltpu.async_copy(y_hbm.at[i0_v.at[0, pl.ds(half, half)]], buf0b,
                           sem2)
    db1 = pltpu.async_copy(y_hbm.at[i1_v.at[0, pl.ds(half, half)]], buf1b,
                           sem3)
    nseg = c_dim // SCL

    def add_rows(b0, b1):
        def body(r, carry):
            for cseg in range(nseg):
                s = b0[r, pl.ds(cseg * SCL, SCL)] + b1[r, pl.ds(cseg * SCL,
                                                                SCL)]
                b0[r, pl.ds(cseg * SCL, SCL)] = s
            return carry
        lax.fori_loop(0, half, body, 0)

    da0.wait()
    da1.wait()
    add_rows(buf0a, buf1a)
    doa = pltpu.async_copy(buf0a,
                           out_hbm.at[pl.ds(wid * tokens_per_worker, half)],
                           soa)
    db0.wait()
    db1.wait()
    add_rows(buf0b, buf1b)
    dob = pltpu.async_copy(
        buf0b, out_hbm.at[pl.ds(wid * tokens_per_worker + half, half)], sob)
    doa.wait()
    dob.wait()


def kernel(x, gate_W, gate_b, expert_W1, expert_b1, expert_W2, expert_b2):
    B, T, C = x.shape
    E = expert_W1.shape[0]
    H = expert_W1.shape[2]
    N = B * T
    NP = N * TOP_K
    NPAD = NP + E * BLK
    NBLK = NPAD // BLK
    NBLK_PAD = ((NBLK + SCL - 1) // SCL) * SCL
    x_flat = x.reshape(N, C)

    gate_logits = x_flat @ gate_W + gate_b
    lg_p = jnp.pad(gate_logits, ((0, 0), (0, LANES - E)),
                   constant_values=-1e30)
    gate_out = pl.pallas_call(
        _gate_body,
        out_shape=jax.ShapeDtypeStruct((N, LANES), jnp.float32),
    )(lg_p)
    tok_pw = N // NW
    eid2 = (gate_out[:, 2:2 + TOP_K].astype(jnp.int32)
            .reshape(NW, tok_pw, TOP_K).transpose(0, 2, 1)
            .reshape(NP // SCL, SCL))
    wgt2 = (gate_out[:, :TOP_K]
            .reshape(NW, tok_pw, TOP_K).transpose(0, 2, 1)
            .reshape(NP // SCL, SCL))

    mesh = plsc.VectorSubcoreMesh(core_axis_name="c", subcore_axis_name="s",
                                  num_cores=SC_CORES, num_subcores=SC_SUBCORES)
    n_chunks = NP // SCL
    route = pl.kernel(
        functools.partial(_sc_route, tok_pw=tok_pw, n_chunks=n_chunks,
                          nblk=NBLK, nblk_pad=NBLK_PAD),
        mesh=mesh,
        compiler_params=pltpu.CompilerParams(needs_layout_passes=False),
        out_type=(
            jax.ShapeDtypeStruct((NPAD, C), jnp.float32),
            jax.ShapeDtypeStruct((NPAD, WCOL), jnp.float32),
            jax.ShapeDtypeStruct((NW, tok_pw), jnp.int32),
            jax.ShapeDtypeStruct((NW, tok_pw), jnp.int32),
            jax.ShapeDtypeStruct((NBLK_PAD,), jnp.int32),
        ),
        scratch_types=[
            pltpu.VMEM((n_chunks, SCL), jnp.int32),
            pltpu.VMEM((n_chunks, SCL), jnp.float32),
            pltpu.VMEM((tok_pw, C), jnp.float32),
            pltpu.VMEM((TOP_K, tok_pw), jnp.int32),
            pltpu.VMEM((tok_pw, WCOL), jnp.float32),
            pltpu.VMEM((NBLK_PAD,), jnp.int32),
            pltpu.SemaphoreType.DMA,
            pltpu.SemaphoreType.DMA,
            pltpu.SemaphoreType.DMA,
            pltpu.SemaphoreType.DMA,
            pltpu.SemaphoreType.DMA,
            pltpu.SemaphoreType.DMA,
        ],
    )(x_flat, eid2, wgt2)
    x_sorted, wcol, p0, p1, block_expert = route

    hch = H // H_CHUNKS
    grid_spec = pltpu.PrefetchScalarGridSpec(
        num_scalar_prefetch=1,
        grid=(NBLK, H_CHUNKS),
        in_specs=[
            pl.BlockSpec((BLK, C), lambda b, hc, be: (b, 0)),
            pl.BlockSpec((1, C, hch), lambda b, hc, be: (be[b], 0, hc)),
            pl.BlockSpec((1, 1, hch), lambda b, hc, be: (be[b], 0, hc)),
            pl.BlockSpec((1, hch, C), lambda b, hc, be: (be[b], hc, 0)),
            pl.BlockSpec((1, 1, C), lambda b, hc, be: (be[b], 0, 0)),
            pl.BlockSpec((BLK, WCOL), lambda b, hc, be: (b, 0)),
        ],
        out_specs=pl.BlockSpec((BLK, C), lambda b, hc, be: (b, 0)),
        scratch_shapes=[pltpu.VMEM((BLK, C), jnp.float32)],
    )
    y_sorted = pl.pallas_call(
        _ffn_body,
        grid_spec=grid_spec,
        out_shape=jax.ShapeDtypeStruct((NPAD, C), jnp.float32),
    )(block_expert, x_sorted, expert_W1, expert_b1.reshape(E, 1, H),
      expert_W2, expert_b2.reshape(E, 1, C), wcol)

    out_flat = pl.kernel(
        functools.partial(_sc_combine, tokens_per_worker=tok_pw, c_dim=C),
        mesh=mesh,
        out_type=jax.ShapeDtypeStruct((N, C), jnp.float32),
        scratch_types=[
            pltpu.VMEM((1, tok_pw), jnp.int32),
            pltpu.VMEM((1, tok_pw), jnp.int32),
            pltpu.VMEM((tok_pw // 2, C), jnp.float32),
            pltpu.VMEM((tok_pw // 2, C), jnp.float32),
            pltpu.VMEM((tok_pw // 2, C), jnp.float32),
            pltpu.VMEM((tok_pw // 2, C), jnp.float32),
            pltpu.SemaphoreType.DMA,
            pltpu.SemaphoreType.DMA,
            pltpu.SemaphoreType.DMA,
            pltpu.SemaphoreType.DMA,
            pltpu.SemaphoreType.DMA,
            pltpu.SemaphoreType.DMA,
        ],
    )(y_sorted, p0, p1)

    return out_flat.reshape(B, T, C)
